# Initial kernel scaffold; baseline (speedup 1.0000x reference)
import jax
import jax.numpy as jnp
from jax.experimental import pallas as pl


def kernel(x, y, q_weight, q_bias, k_weight, k_bias):
    raise NotImplementedError("write your pallas kernel here")


# speedup vs baseline: 19.6328x; 19.6328x over previous
import jax
import jax.numpy as jnp
import numpy as np
from jax.experimental import pallas as pl

_DIM = 96
_TOPK = 4
_H = 57
_W = 57
_RS = 8
_NR = 8
_GP = _NR * _RS
_SCALE = _DIM ** (-0.5)


def _np_constants():
    Gh = np.zeros((_NR, _GP), np.float32)
    Gh[np.arange(_NR), np.arange(_NR) * _RS] = 1.0
    Gw = np.zeros((_NR * _TOPK, _GP), np.float32)
    for j in range(_NR):
        for t in range(_TOPK):
            Gw[j * _TOPK + t, j * _RS + t] = 1.0
    C = (np.arange(_GP)[:, None] % _RS == np.arange(_GP)[None, :] % _RS)
    return Gh, Gw, C.astype(np.float32)


def _avgpool_ceil(x, kh, kw):
    n, c, h, w = x.shape
    oh = -(-h // kh)
    ow = -(-w // kw)
    xp = jnp.pad(x, ((0, 0), (0, 0), (0, oh * kh - h), (0, ow * kw - w)))
    s = xp.reshape(n, c, oh, kh, ow, kw).sum(axis=(3, 5))
    vh = np.minimum(np.arange(oh) * kh + kh, h) - np.arange(oh) * kh
    vw = np.minimum(np.arange(ow) * kw + kw, w) - np.arange(ow) * kw
    cnt = (vh[:, None] * vw[None, :]).astype(np.float32)
    return s / jnp.asarray(cnt)[None, None, :, :]


def _prep_kernel(x_ref, y_ref, qw_ref, qb_ref, kw_ref, kb_ref,
                 ar_ref, gh_ref, gw_ref,
                 a3t_ref, idx_ref, f_ref):
    x = x_ref[0]
    y = y_ref[0]
    qw = qw_ref[...]
    kw = kw_ref[...]
    qb = qb_ref[...]
    kb = kb_ref[...]
    ar = ar_ref[0]
    mrow = jax.lax.broadcasted_iota(jnp.int32, (_GP, _GP), 0) < _H
    mcol = jax.lax.broadcasted_iota(jnp.int32, (_GP, _GP), 1) < _W
    mask = (mrow & mcol).astype(jnp.float32)[None]
    dn = (((1,), (0,)), ((), ()))
    q = jax.lax.dot_general(qw, x, dn, preferred_element_type=jnp.float32)
    qp = (q + qb[0][:, None, None]) * mask
    k = jax.lax.dot_general(kw, y, dn, preferred_element_type=jnp.float32)
    kp = (k + kb[0][:, None, None]) * mask
    iv = jax.lax.broadcasted_iota(jnp.int32, (_GP, _GP), 1)
    cur = ar
    neg = jnp.float32(-jnp.inf)
    for t in range(_TOPK):
        m = jnp.max(cur, axis=1, keepdims=True)
        cand = jnp.where(cur >= m, iv, 127)
        it = jnp.min(cand, axis=1).astype(jnp.int32)
        idx_ref[0, :, 0, t] = it
        hit = iv == it[:, None]
        cur = jnp.where(hit, neg, cur)
    for qi in range(_NR):
        a3t_ref[0, qi] = _SCALE * ar[:, qi * _RS:(qi + 1) * _RS]
    dn1 = (((1,), (1,)), ((), ()))
    Gh = gh_ref[...]
    Gw = gw_ref[...]
    u1 = jax.lax.dot_general(qp, Gh, dn1,
                             preferred_element_type=jnp.float32)
    qpix = jax.lax.dot_general(u1, Gw, dn1,
                               preferred_element_type=jnp.float32)
    F = jax.lax.dot_general(qpix, kp, (((0,), (0,)), ((), ())),
                            preferred_element_type=jnp.float32)
    f_ref[0] = _SCALE * F


def _out_kernel(a3t_ref, idx_ref, f_ref, c_ref, o_ref):
    Arow = a3t_ref[0, 0]
    ones = jnp.ones((1, _GP), jnp.float32)
    C = c_ref[...]
    R = (8 * (jax.lax.broadcasted_iota(jnp.int32, (_GP, _GP), 0) // _RS)
         + jax.lax.broadcasted_iota(jnp.int32, (_GP, _GP), 1) // _RS)
    n = 8 * pl.program_id(1) + pl.program_id(2)
    r3 = idx_ref[0, n, 0, 3]
    ri3 = r3 // 8
    rj3 = r3 % 8
    col = jax.lax.broadcasted_iota(jnp.int32, (1, _GP), 1) // _RS
    Th = C * (col == ri3).astype(jnp.float32)
    Tw = C * (col == rj3).astype(jnp.float32)
    slab0 = None
    for pi in range(_NR):
        colv = Arow[:, pi:pi + 1]
        slab = jax.lax.dot_general(colv, ones, (((1,), (0,)), ((), ())),
                                   preferred_element_type=jnp.float32)
        if pi == 0:
            slab0 = slab
        o_ref[0, pi] = jnp.broadcast_to(slab[None, :_H, :_W], (_RS, _H, _W))
    for t in range(_TOPK):
        rt = idx_ref[0, n, 0, t]
        Ft = f_ref[0, 0, t]
        v1 = jax.lax.dot_general(Th, Ft, (((1,), (0,)), ((), ())),
                                 preferred_element_type=jnp.float32)
        V = jax.lax.dot_general(v1, Tw, (((1,), (1,)), ((), ())),
                                preferred_element_type=jnp.float32)
        slab = jnp.where(R == rt, V, slab0)
        o_ref[0, 0, t] = slab[:_H, :_W]


@jax.jit
def kernel(x, y, q_weight, q_bias, k_weight, k_bias):
    B = x.shape[0]
    Gh, Gw, C = (jnp.asarray(a) for a in _np_constants())
    qw = q_weight[:, :, 0, 0]
    kw = k_weight[:, :, 0, 0]
    qb = q_bias[None, :]
    kb = k_bias[None, :]

    q = jnp.einsum('oc,nchw->nohw', qw, x) + q_bias[None, :, None, None]
    k = jnp.einsum('oc,nchw->nohw', kw, y) + k_bias[None, :, None, None]
    q_r = _avgpool_ceil(jax.lax.stop_gradient(q), _RS, _RS)
    k_r = _avgpool_ceil(jax.lax.stop_gradient(k), _RS, _RS)
    nb, c, h, w = q_r.shape
    q_rf = q_r.transpose(0, 2, 3, 1).reshape(B, h * w, c)
    k_rf = k_r.reshape(B, c, h * w)
    a_r = jnp.einsum('bnc,bck->bnk', q_rf, k_rf)

    xp = jnp.pad(x, ((0, 0), (0, 0), (0, _GP - _H), (0, _GP - _W)))
    yp = jnp.pad(y, ((0, 0), (0, 0), (0, _GP - _H), (0, _GP - _W)))

    a3t, idx4, F = pl.pallas_call(
        _prep_kernel,
        grid=(B,),
        in_specs=[
            pl.BlockSpec((1, _DIM, _GP, _GP), lambda b: (b, 0, 0, 0)),
            pl.BlockSpec((1, _DIM, _GP, _GP), lambda b: (b, 0, 0, 0)),
            pl.BlockSpec((_DIM, _DIM), lambda b: (0, 0)),
            pl.BlockSpec((1, _DIM), lambda b: (0, 0)),
            pl.BlockSpec((_DIM, _DIM), lambda b: (0, 0)),
            pl.BlockSpec((1, _DIM), lambda b: (0, 0)),
            pl.BlockSpec((1, _GP, _GP), lambda b: (b, 0, 0)),
            pl.BlockSpec((_NR, _GP), lambda b: (0, 0)),
            pl.BlockSpec((_NR * _TOPK, _GP), lambda b: (0, 0)),
        ],
        out_specs=[
            pl.BlockSpec((1, _NR, _GP, _NR), lambda b: (b, 0, 0, 0)),
            pl.BlockSpec((1, _GP, 1, _TOPK), lambda b: (b, 0, 0, 0)),
            pl.BlockSpec((1, _NR, _NR * _TOPK, _GP, _GP),
                         lambda b: (b, 0, 0, 0, 0)),
        ],
        out_shape=[
            jax.ShapeDtypeStruct((B, _NR, _GP, _NR), jnp.float32),
            jax.ShapeDtypeStruct((B, _GP, 1, _TOPK), jnp.int32),
            jax.ShapeDtypeStruct((B, _NR, _NR * _TOPK, _GP, _GP), jnp.float32),
        ],
    )(xp, yp, qw, qb, kw, kb, a_r, Gh, Gw)

    out = pl.pallas_call(
        _out_kernel,
        grid=(B, _NR, _NR),
        in_specs=[
            pl.BlockSpec((1, 1, _GP, _NR), lambda b, i, j: (b, i, 0, 0)),
            pl.BlockSpec((1, _GP, 1, _TOPK), lambda b, i, j: (b, 0, 0, 0)),
            pl.BlockSpec((1, 1, _TOPK, _GP, _GP),
                         lambda b, i, j: (b, i, j, 0, 0)),
            pl.BlockSpec((_GP, _GP), lambda b, i, j: (0, 0)),
        ],
        out_specs=pl.BlockSpec((1, _RS, _RS, _H, _W),
                               lambda b, i, j: (b, i, j, 0, 0)),
        out_shape=jax.ShapeDtypeStruct((B, _H, _W, _H, _W), jnp.float32),
    )(a3t, idx4, F, C)

    return out[:, :, :, None, :, :]
